# Initial kernel scaffold; baseline (speedup 1.0000x reference)
import functools

import jax
import jax.numpy as jnp
from jax import lax
from jax.experimental import pallas as pl
from jax.experimental.pallas import tpu as pltpu
from jax.experimental.pallas import tpu_sc as plsc

N = 50000
B = 128
H = 256
NC, NS = 2, 16
NW = NC * NS
C = 128
FULL = N // C
TAIL = N - FULL * C
ITERS = (FULL + NW - 1) // NW
ROWS_PER_TILE = B // NS

_mesh = plsc.VectorSubcoreMesh(core_axis_name="c", subcore_axis_name="s")


@functools.partial(
    pl.kernel,
    out_type=jax.ShapeDtypeStruct((NC, 2, B, H), jnp.float32),
    mesh=_mesh,
    scratch_types=[
        pltpu.VMEM((C,), jnp.int32),
        pltpu.VMEM((TAIL,), jnp.int32),
        pltpu.VMEM((C, H), jnp.float32),
        pltpu.VMEM((C, H), jnp.float32),
        pltpu.VMEM((B // NS, H), jnp.float32),
        pltpu.VMEM_SHARED((B, H), jnp.float32),
        pltpu.VMEM_SHARED((B, H), jnp.float32),
    ],
)
def _segment_sums_sc(node_hbm, edge_hbm, idx_hbm, out_hbm,
                     idx_v, idxt_v, nrows_v, erows_v, buf_v, accn_s, acce_s):
    cid = lax.axis_index("c")
    sid = lax.axis_index("s")
    wid = sid * NC + cid

    zero = jnp.zeros((16,), jnp.float32)
    for r in range(ROWS_PER_TILE):
        for j in range(H // 16):
            buf_v[r, pl.ds(j * 16, 16)] = zero
    pltpu.sync_copy(buf_v, accn_s.at[pl.ds(sid * ROWS_PER_TILE, ROWS_PER_TILE)])
    pltpu.sync_copy(buf_v, acce_s.at[pl.ds(sid * ROWS_PER_TILE, ROWS_PER_TILE)])
    plsc.subcore_barrier()

    def chunk_body(i, carry):
        k = wid + i * NW
        @pl.when(k < FULL)
        def _():
            base = k * C
            pltpu.sync_copy(idx_hbm.at[pl.ds(base, C)], idx_v)
            pltpu.sync_copy(node_hbm.at[pl.ds(base, C)], nrows_v)
            pltpu.sync_copy(edge_hbm.at[pl.ds(base, C)], erows_v)
            pltpu.sync_copy(nrows_v, accn_s.at[idx_v], add=True)
            pltpu.sync_copy(erows_v, acce_s.at[idx_v], add=True)
        return carry

    lax.fori_loop(0, ITERS, chunk_body, 0)

    @pl.when(wid == NW - 1)
    def _():
        base = FULL * C
        pltpu.sync_copy(idx_hbm.at[pl.ds(base, TAIL)], idxt_v)
        pltpu.sync_copy(node_hbm.at[pl.ds(base, TAIL)], nrows_v.at[pl.ds(0, TAIL)])
        pltpu.sync_copy(edge_hbm.at[pl.ds(base, TAIL)], erows_v.at[pl.ds(0, TAIL)])
        pltpu.sync_copy(nrows_v.at[pl.ds(0, TAIL)], accn_s.at[idxt_v], add=True)
        pltpu.sync_copy(erows_v.at[pl.ds(0, TAIL)], acce_s.at[idxt_v], add=True)

    plsc.subcore_barrier()

    r0 = sid * ROWS_PER_TILE
    pltpu.sync_copy(accn_s.at[pl.ds(r0, ROWS_PER_TILE)], buf_v)
    pltpu.sync_copy(buf_v, out_hbm.at[cid, 0, pl.ds(r0, ROWS_PER_TILE)])
    pltpu.sync_copy(acce_s.at[pl.ds(r0, ROWS_PER_TILE)], buf_v)
    pltpu.sync_copy(buf_v, out_hbm.at[cid, 1, pl.ds(r0, ROWS_PER_TILE)])


def _softplus(x):
    return jnp.maximum(x, 0.0) + jnp.log1p(jnp.exp(-jnp.abs(x)))


def _mlp_tc(g_ref, p_ref, wc_ref, bc_ref, w1_ref, b1_ref, w2_ref, b2_ref,
            w3_ref, b3_ref, out_ref):
    na = p_ref[0, 0] + p_ref[1, 0]
    ea = p_ref[0, 1] + p_ref[1, 1]
    wc = wc_ref[...]
    dot = functools.partial(jnp.dot, preferred_element_type=jnp.float32,
                            precision=lax.Precision.HIGHEST)
    comb = (dot(g_ref[...], wc[0:H])
            + dot(na, wc[H:2 * H])
            + dot(ea, wc[2 * H:3 * H])
            + bc_ref[...])
    h = _softplus(dot(comb, w1_ref[...]) + b1_ref[...])
    h = _softplus(dot(h, w2_ref[...]) + b2_ref[...])
    out_ref[...] = dot(h, w3_ref[...]) + b3_ref[...]


def kernel(global_feat, node_features, edge_features, batch_idx,
           W_c, b_c, W1, b1, W2, b2, W3, b3):
    idx = batch_idx.astype(jnp.int32)
    partials = _segment_sums_sc(node_features, edge_features, idx)
    out = pl.pallas_call(
        _mlp_tc,
        out_shape=jax.ShapeDtypeStruct((B, H), jnp.float32),
    )(global_feat, partials,
      W_c, b_c.reshape(1, H), W1, b1.reshape(1, H),
      W2, b2.reshape(1, H), W3, b3.reshape(1, H))
    return out


# speedup vs baseline: 2.7726x; 2.7726x over previous
import functools

import jax
import jax.numpy as jnp
from jax import lax
from jax.experimental import pallas as pl
from jax.experimental.pallas import tpu as pltpu
from jax.experimental.pallas import tpu_sc as plsc

N = 50000
B = 128
H = 256
NC, NS = 2, 16
NW = NC * NS
C = 96
FULL = N // C
TAIL = N - FULL * C
ITERS = (FULL + NW - 1) // NW

_mesh = plsc.VectorSubcoreMesh(core_axis_name="c", subcore_axis_name="s")


@functools.partial(
    pl.kernel,
    out_type=jax.ShapeDtypeStruct((NW, 2, B, H), jnp.float32),
    mesh=_mesh,
    scratch_types=[
        pltpu.VMEM((C,), jnp.int32),
        pltpu.VMEM((C, H), jnp.float32),
        pltpu.VMEM((C, H), jnp.float32),
        pltpu.VMEM((B, H), jnp.float32),
        pltpu.VMEM((B, H), jnp.float32),
    ],
)
def _segment_sums_sc(node_hbm, edge_hbm, idx_hbm, out_hbm,
                     idx_v, nrows_v, erows_v, accn_v, acce_v):
    cid = lax.axis_index("c")
    sid = lax.axis_index("s")
    wid = sid * NC + cid

    zero = jnp.zeros((16,), jnp.float32)
    def zero_body(r, carry):
        for j in range(H // 16):
            accn_v[r, pl.ds(j * 16, 16)] = zero
            acce_v[r, pl.ds(j * 16, 16)] = zero
        return carry
    lax.fori_loop(0, B, zero_body, 0)

    def accumulate(rows_ref, acc_ref, nrows):
        def grp_body(g, carry):
            segs = idx_v[pl.ds(g * 16, 16)]
            for l in range(16):
                seg = segs[l]
                r = g * 16 + l
                for j in range(H // 16):
                    val = rows_ref[r, pl.ds(j * 16, 16)]
                    plsc.addupdate(acc_ref.at[seg, pl.ds(j * 16, 16)], val)
            return carry
        lax.fori_loop(0, nrows // 16, grp_body, 0)

    def chunk_body(i, carry):
        k = wid + i * NW
        @pl.when(k < FULL)
        def _():
            base = k * C
            pltpu.sync_copy(idx_hbm.at[pl.ds(base, C)], idx_v)
            pltpu.sync_copy(node_hbm.at[pl.ds(base, C)], nrows_v)
            pltpu.sync_copy(edge_hbm.at[pl.ds(base, C)], erows_v)
            accumulate(nrows_v, accn_v, C)
            accumulate(erows_v, acce_v, C)
        return carry

    lax.fori_loop(0, ITERS, chunk_body, 0)

    @pl.when(wid == NW - 1)
    def _():
        base = FULL * C
        pltpu.sync_copy(idx_hbm.at[pl.ds(base, TAIL)], idx_v.at[pl.ds(0, TAIL)])
        pltpu.sync_copy(node_hbm.at[pl.ds(base, TAIL)], nrows_v.at[pl.ds(0, TAIL)])
        pltpu.sync_copy(edge_hbm.at[pl.ds(base, TAIL)], erows_v.at[pl.ds(0, TAIL)])
        accumulate(nrows_v, accn_v, TAIL)
        accumulate(erows_v, acce_v, TAIL)

    pltpu.sync_copy(accn_v, out_hbm.at[wid, 0])
    pltpu.sync_copy(acce_v, out_hbm.at[wid, 1])


def _softplus(x):
    return jnp.maximum(x, 0.0) + jnp.log1p(jnp.exp(-jnp.abs(x)))


def _mlp_tc(g_ref, p_ref, wc_ref, bc_ref, w1_ref, b1_ref, w2_ref, b2_ref,
            w3_ref, b3_ref, out_ref):
    na = jnp.sum(p_ref[:, 0], axis=0)
    ea = jnp.sum(p_ref[:, 1], axis=0)
    wc = wc_ref[...]
    dot = functools.partial(jnp.dot, preferred_element_type=jnp.float32,
                            precision=lax.Precision.HIGHEST)
    comb = (dot(g_ref[...], wc[0:H])
            + dot(na, wc[H:2 * H])
            + dot(ea, wc[2 * H:3 * H])
            + bc_ref[...])
    h = _softplus(dot(comb, w1_ref[...]) + b1_ref[...])
    h = _softplus(dot(h, w2_ref[...]) + b2_ref[...])
    out_ref[...] = dot(h, w3_ref[...]) + b3_ref[...]


def kernel(global_feat, node_features, edge_features, batch_idx,
           W_c, b_c, W1, b1, W2, b2, W3, b3):
    idx = batch_idx.astype(jnp.int32)
    partials = _segment_sums_sc(node_features, edge_features, idx)
    out = pl.pallas_call(
        _mlp_tc,
        out_shape=jax.ShapeDtypeStruct((B, H), jnp.float32),
    )(global_feat, partials,
      W_c, b_c.reshape(1, H), W1, b1.reshape(1, H),
      W2, b2.reshape(1, H), W3, b3.reshape(1, H))
    return out


# speedup vs baseline: 3.4727x; 1.2525x over previous
import functools

import jax
import jax.numpy as jnp
from jax import lax
from jax.experimental import pallas as pl
from jax.experimental.pallas import tpu as pltpu
from jax.experimental.pallas import tpu_sc as plsc

N = 50000
B = 128
H = 256
NC, NS = 2, 16
NW = NC * NS
C = 48
FULL = N // C
TAIL = N - FULL * C
ITERS = (FULL + NW - 1) // NW
OUTER = (ITERS + 1) // 2

_mesh = plsc.VectorSubcoreMesh(core_axis_name="c", subcore_axis_name="s")


@functools.partial(
    pl.kernel,
    out_type=jax.ShapeDtypeStruct((NW, 2, B, H), jnp.float32),
    mesh=_mesh,
    scratch_types=[
        pltpu.VMEM((C,), jnp.int32),
        pltpu.VMEM((C,), jnp.int32),
        pltpu.VMEM((C, H), jnp.float32),
        pltpu.VMEM((C, H), jnp.float32),
        pltpu.VMEM((C, H), jnp.float32),
        pltpu.VMEM((C, H), jnp.float32),
        pltpu.VMEM((B, H), jnp.float32),
        pltpu.VMEM((B, H), jnp.float32),
        pltpu.SemaphoreType.DMA,
        pltpu.SemaphoreType.DMA,
        pltpu.SemaphoreType.DMA,
        pltpu.SemaphoreType.DMA,
        pltpu.SemaphoreType.DMA,
        pltpu.SemaphoreType.DMA,
    ],
)
def _segment_sums_sc(node_hbm, edge_hbm, idx_hbm, out_hbm,
                     idx0_v, idx1_v, n0_v, n1_v, e0_v, e1_v,
                     accn_v, acce_v,
                     si0, si1, sn0, sn1, se0, se1):
    cid = lax.axis_index("c")
    sid = lax.axis_index("s")
    wid = sid * NC + cid

    idx_bufs = (idx0_v, idx1_v)
    n_bufs = (n0_v, n1_v)
    e_bufs = (e0_v, e1_v)
    sems = ((si0, sn0, se0), (si1, sn1, se1))

    zero = jnp.zeros((16,), jnp.float32)
    def zero_body(r, carry):
        for j in range(H // 16):
            accn_v[r, pl.ds(j * 16, 16)] = zero
            acce_v[r, pl.ds(j * 16, 16)] = zero
        return carry
    lax.fori_loop(0, B, zero_body, 0)

    def issue(i, b):
        k = wid + i * NW
        @pl.when(k < FULL)
        def _():
            base = k * C
            pltpu.async_copy(idx_hbm.at[pl.ds(base, C)], idx_bufs[b], sems[b][0])
            pltpu.async_copy(node_hbm.at[pl.ds(base, C)], n_bufs[b], sems[b][1])
            pltpu.async_copy(edge_hbm.at[pl.ds(base, C)], e_bufs[b], sems[b][2])

    def wait(i, b):
        k = wid + i * NW
        @pl.when(k < FULL)
        def _():
            pltpu.make_async_copy(idx_hbm.at[pl.ds(0, C)], idx_bufs[b], sems[b][0]).wait()
            pltpu.make_async_copy(node_hbm.at[pl.ds(0, C)], n_bufs[b], sems[b][1]).wait()
            pltpu.make_async_copy(edge_hbm.at[pl.ds(0, C)], e_bufs[b], sems[b][2]).wait()

    def accumulate(idx_ref, rows_ref, acc_ref, nrows):
        def grp_body(g, carry):
            segs = idx_ref[pl.ds(g * 16, 16)]
            for l in range(16):
                seg = segs[l]
                r = g * 16 + l
                for j in range(H // 16):
                    val = rows_ref[r, pl.ds(j * 16, 16)]
                    plsc.addupdate(acc_ref.at[seg, pl.ds(j * 16, 16)], val)
            return carry
        lax.fori_loop(0, nrows // 16, grp_body, 0)

    issue(0, 0)
    issue(1, 1)

    def outer_body(t, carry):
        for b in range(2):
            i = 2 * t + b
            k = wid + i * NW
            wait(i, b)
            @pl.when(k < FULL)
            def _():
                accumulate(idx_bufs[b], n_bufs[b], accn_v, C)
                accumulate(idx_bufs[b], e_bufs[b], acce_v, C)
            issue(i + 2, b)
        return carry

    lax.fori_loop(0, OUTER, outer_body, 0)

    @pl.when(wid == NW - 1)
    def _():
        base = FULL * C
        pltpu.sync_copy(idx_hbm.at[pl.ds(base, TAIL)], idx0_v.at[pl.ds(0, TAIL)])
        pltpu.sync_copy(node_hbm.at[pl.ds(base, TAIL)], n0_v.at[pl.ds(0, TAIL)])
        pltpu.sync_copy(edge_hbm.at[pl.ds(base, TAIL)], e0_v.at[pl.ds(0, TAIL)])
        accumulate(idx0_v, n0_v, accn_v, TAIL)
        accumulate(idx0_v, e0_v, acce_v, TAIL)

    pltpu.sync_copy(accn_v, out_hbm.at[wid, 0])
    pltpu.sync_copy(acce_v, out_hbm.at[wid, 1])


def _softplus(x):
    return jnp.maximum(x, 0.0) + jnp.log1p(jnp.exp(-jnp.abs(x)))


def _mlp_tc(g_ref, p_ref, wc_ref, bc_ref, w1_ref, b1_ref, w2_ref, b2_ref,
            w3_ref, b3_ref, out_ref):
    na = jnp.sum(p_ref[:, 0], axis=0)
    ea = jnp.sum(p_ref[:, 1], axis=0)
    wc = wc_ref[...]
    dot = functools.partial(jnp.dot, preferred_element_type=jnp.float32,
                            precision=lax.Precision.HIGHEST)
    comb = (dot(g_ref[...], wc[0:H])
            + dot(na, wc[H:2 * H])
            + dot(ea, wc[2 * H:3 * H])
            + bc_ref[...])
    h = _softplus(dot(comb, w1_ref[...]) + b1_ref[...])
    h = _softplus(dot(h, w2_ref[...]) + b2_ref[...])
    out_ref[...] = dot(h, w3_ref[...]) + b3_ref[...]


def kernel(global_feat, node_features, edge_features, batch_idx,
           W_c, b_c, W1, b1, W2, b2, W3, b3):
    idx = batch_idx.astype(jnp.int32)
    partials = _segment_sums_sc(node_features, edge_features, idx)
    out = pl.pallas_call(
        _mlp_tc,
        out_shape=jax.ShapeDtypeStruct((B, H), jnp.float32),
    )(global_feat, partials,
      W_c, b_c.reshape(1, H), W1, b1.reshape(1, H),
      W2, b2.reshape(1, H), W3, b3.reshape(1, H))
    return out


# speedup vs baseline: 6.4302x; 1.8516x over previous
import functools

import jax
import jax.numpy as jnp
from jax import lax
from jax.experimental import pallas as pl
from jax.experimental.pallas import tpu as pltpu
from jax.experimental.pallas import tpu_sc as plsc

N = 50000
B = 128
H = 256
NC, NS = 2, 16
NW = NC * NS
C = 48
FULL = N // C
TAIL = N - FULL * C
ITERS = (FULL + NW - 1) // NW
OUTER = (ITERS + 1) // 2

_mesh = plsc.VectorSubcoreMesh(core_axis_name="c", subcore_axis_name="s")


@functools.partial(
    pl.kernel,
    out_type=jax.ShapeDtypeStruct((NW, 2, B, H), jnp.float32),
    mesh=_mesh,
    scratch_types=[
        pltpu.VMEM((C,), jnp.int32),
        pltpu.VMEM((C,), jnp.int32),
        pltpu.VMEM((C, H), jnp.float32),
        pltpu.VMEM((C, H), jnp.float32),
        pltpu.VMEM((C, H), jnp.float32),
        pltpu.VMEM((C, H), jnp.float32),
        pltpu.VMEM((B, H), jnp.float32),
        pltpu.VMEM((B, H), jnp.float32),
        pltpu.SemaphoreType.DMA,
        pltpu.SemaphoreType.DMA,
        pltpu.SemaphoreType.DMA,
        pltpu.SemaphoreType.DMA,
        pltpu.SemaphoreType.DMA,
        pltpu.SemaphoreType.DMA,
    ],
)
def _segment_sums_sc(node_hbm, edge_hbm, idx_hbm, out_hbm,
                     idx0_v, idx1_v, n0_v, n1_v, e0_v, e1_v,
                     accn_v, acce_v,
                     si0, si1, sn0, sn1, se0, se1):
    cid = lax.axis_index("c")
    sid = lax.axis_index("s")
    wid = sid * NC + cid

    idx_bufs = (idx0_v, idx1_v)
    n_bufs = (n0_v, n1_v)
    e_bufs = (e0_v, e1_v)
    sems = ((si0, sn0, se0), (si1, sn1, se1))

    zero = jnp.zeros((16,), jnp.float32)
    def zero_body(r, carry):
        for j in range(H // 16):
            accn_v[r, pl.ds(j * 16, 16)] = zero
            acce_v[r, pl.ds(j * 16, 16)] = zero
        return carry
    lax.fori_loop(0, B, zero_body, 0)

    def issue(i, b):
        k = wid + i * NW
        @pl.when(k < FULL)
        def _():
            base = k * C
            pltpu.async_copy(idx_hbm.at[pl.ds(base, C)], idx_bufs[b], sems[b][0])
            pltpu.async_copy(node_hbm.at[pl.ds(base, C)], n_bufs[b], sems[b][1])
            pltpu.async_copy(edge_hbm.at[pl.ds(base, C)], e_bufs[b], sems[b][2])

    def wait(i, b):
        k = wid + i * NW
        @pl.when(k < FULL)
        def _():
            pltpu.make_async_copy(idx_hbm.at[pl.ds(0, C)], idx_bufs[b], sems[b][0]).wait()
            pltpu.make_async_copy(node_hbm.at[pl.ds(0, C)], n_bufs[b], sems[b][1]).wait()
            pltpu.make_async_copy(edge_hbm.at[pl.ds(0, C)], e_bufs[b], sems[b][2]).wait()

    def accumulate(idx_ref, rows_ref, acc_ref, nrows):
        NJ = H // 16

        def grp_body(g, carry):
            segs = idx_ref[pl.ds(g * 16, 16)]
            for l in range(16):
                seg = segs[l]
                r = g * 16 + l
                vals = [rows_ref[r, pl.ds(j * 16, 16)] for j in range(NJ)]
                for j in range(NJ):
                    plsc.addupdate(acc_ref.at[seg, pl.ds(j * 16, 16)], vals[j])
            return carry
        lax.fori_loop(0, nrows // 16, grp_body, 0)

    issue(0, 0)
    issue(1, 1)

    def outer_body(t, carry):
        for b in range(2):
            i = 2 * t + b
            k = wid + i * NW
            wait(i, b)
            @pl.when(k < FULL)
            def _():
                accumulate(idx_bufs[b], n_bufs[b], accn_v, C)
                accumulate(idx_bufs[b], e_bufs[b], acce_v, C)
            issue(i + 2, b)
        return carry

    lax.fori_loop(0, OUTER, outer_body, 0)

    @pl.when(wid == NW - 1)
    def _():
        base = FULL * C
        pltpu.sync_copy(idx_hbm.at[pl.ds(base, TAIL)], idx0_v.at[pl.ds(0, TAIL)])
        pltpu.sync_copy(node_hbm.at[pl.ds(base, TAIL)], n0_v.at[pl.ds(0, TAIL)])
        pltpu.sync_copy(edge_hbm.at[pl.ds(base, TAIL)], e0_v.at[pl.ds(0, TAIL)])
        accumulate(idx0_v, n0_v, accn_v, TAIL)
        accumulate(idx0_v, e0_v, acce_v, TAIL)

    pltpu.sync_copy(accn_v, out_hbm.at[wid, 0])
    pltpu.sync_copy(acce_v, out_hbm.at[wid, 1])


def _softplus(x):
    return jnp.maximum(x, 0.0) + jnp.log1p(jnp.exp(-jnp.abs(x)))


def _mlp_tc(g_ref, p_ref, wc_ref, bc_ref, w1_ref, b1_ref, w2_ref, b2_ref,
            w3_ref, b3_ref, out_ref):
    na = jnp.sum(p_ref[:, 0], axis=0)
    ea = jnp.sum(p_ref[:, 1], axis=0)
    wc = wc_ref[...]
    dot = functools.partial(jnp.dot, preferred_element_type=jnp.float32,
                            precision=lax.Precision.HIGHEST)
    comb = (dot(g_ref[...], wc[0:H])
            + dot(na, wc[H:2 * H])
            + dot(ea, wc[2 * H:3 * H])
            + bc_ref[...])
    h = _softplus(dot(comb, w1_ref[...]) + b1_ref[...])
    h = _softplus(dot(h, w2_ref[...]) + b2_ref[...])
    out_ref[...] = dot(h, w3_ref[...]) + b3_ref[...]


def kernel(global_feat, node_features, edge_features, batch_idx,
           W_c, b_c, W1, b1, W2, b2, W3, b3):
    idx = batch_idx.astype(jnp.int32)
    partials = _segment_sums_sc(node_features, edge_features, idx)
    out = pl.pallas_call(
        _mlp_tc,
        out_shape=jax.ShapeDtypeStruct((B, H), jnp.float32),
    )(global_feat, partials,
      W_c, b_c.reshape(1, H), W1, b1.reshape(1, H),
      W2, b2.reshape(1, H), W3, b3.reshape(1, H))
    return out
